# Initial kernel scaffold; baseline (speedup 1.0000x reference)
import jax
import jax.numpy as jnp
from jax.experimental import pallas as pl


def kernel(Corpus_, batch_inputs, entity_embeddings, relation_embeddings, entity_list, relation_type, entity_list_nhop, relation_type_nhop, W, a0, a20, a1, a21, a_out, a2_out):
    raise NotImplementedError("write your pallas kernel here")


# speedup vs baseline: 1.7093x; 1.7093x over previous
import dataclasses
import functools

import jax
import jax.numpy as jnp
from jax import lax
from jax.experimental import pallas as pl
from jax.experimental.pallas import tpu as pltpu
from jax.experimental.pallas import tpu_sc as plsc

ALPHA = 0.2
C = 64
NCORE = 2
NSUB = 16
RSP = 48
F32 = jnp.float32


def _lrelu(x):
    return jnp.where(x > 0, x, ALPHA * x)


def _elu(x):
    return jnp.where(x > 0, x, jnp.exp(x) - 1.0)


def _dot(a, b):
    return jax.lax.dot(a, b, precision=jax.lax.Precision.HIGHEST,
                       preferred_element_type=F32)


def _blk(shape, index_map):
    return pl.BlockSpec(shape, index_map)


def _tc_pre_body(x_ref, rel_ref, a0_ref, a20_ref, a1_ref, a21_ref,
                 qcat_ref, rcat_ref, pcat_ref,
                 p0_ref, q0_ref, p1_ref, q1_ref, rr0_ref, rr1_ref):
    x = x_ref[...]
    rel = rel_ref[...]
    for h, (a_ref, a2_ref) in enumerate(((a0_ref, a20_ref), (a1_ref, a21_ref))):
        a = a_ref[...]
        a_s = a[:, 0:128]
        a_d = a[:, 128:256]
        a_r = a[:, 256:384]
        a2 = a2_ref[...]
        P = _dot(x, (a_s + a_r).T)
        Q = _dot(x, (a_d + a_r).T)
        Rv = _dot(rel, a_r.T)
        sl = slice(h * 64, (h + 1) * 64)
        pcat_ref[:, sl] = P
        qcat_ref[:, sl] = Q
        rcat_ref[:, sl] = Rv
        (p0_ref if h == 0 else p1_ref)[...] = _dot(P, a2.T)
        (q0_ref if h == 0 else q1_ref)[...] = _dot(Q, a2.T)
        (rr0_ref if h == 0 else rr1_ref)[...] = _dot(Rv, a2.T)


def _tc_pre(br, x_p, rel_p, a0, a20, a1, a21):
    NP = x_p.shape[0]
    RPR = rel_p.shape[0]
    c0 = lambda i: (0, 0)
    nblk = _blk((br, 128), lambda i: (i, 0))
    nsblk = _blk((br, 1), lambda i: (i, 0))
    rblk = _blk((RPR, 128), c0)
    rsblk = _blk((RPR, 1), c0)
    out_shape = (
        jax.ShapeDtypeStruct((NP, 128), F32),
        jax.ShapeDtypeStruct((RPR, 128), F32),
        jax.ShapeDtypeStruct((NP, 128), F32),
        jax.ShapeDtypeStruct((NP, 1), F32),
        jax.ShapeDtypeStruct((NP, 1), F32),
        jax.ShapeDtypeStruct((NP, 1), F32),
        jax.ShapeDtypeStruct((NP, 1), F32),
        jax.ShapeDtypeStruct((RPR, 1), F32),
        jax.ShapeDtypeStruct((RPR, 1), F32),
    )
    return pl.pallas_call(
        _tc_pre_body,
        grid=(NP // br,),
        in_specs=[nblk, rblk, _blk((64, 384), c0), _blk((1, 64), c0),
                  _blk((64, 384), c0), _blk((1, 64), c0)],
        out_specs=(nblk, rblk, nblk, nsblk, nsblk, nsblk, nsblk,
                   rsblk, rsblk),
        out_shape=out_shape)(x_p, rel_p, a0, a20, a1, a21)


def _tc_mid_body(pcat_ref, feats_ref, rs0_ref, rs1_ref, rel_ref, w_ref, a_ref,
                 a2_ref, rel1_ref, p1_ref, pn_ref, q1_ref, qn_ref, rf_ref,
                 p1s_ref, q1s_ref, pns_ref, qns_ref, rrf_ref):
    heads = []
    for h, rs_ref in enumerate((rs0_ref, rs1_ref)):
        rs = rs_ref[...]
        P = pcat_ref[:, h * 64:(h + 1) * 64]
        hacc = feats_ref[:, h * 64:(h + 1) * 64]
        hp = jnp.where(rs > 0, P + hacc / jnp.where(rs > 0, rs, 1.0), 0.0)
        heads.append(_elu(hp))
    x1 = jnp.concatenate(heads, axis=1)
    rel1 = _dot(rel_ref[...], w_ref[...])
    rel1_ref[...] = rel1
    a = a_ref[...]
    a_s = a[:, 0:128]
    a_d = a[:, 128:256]
    a_r = a[:, 256:384]
    a2 = a2_ref[...]
    P1 = _dot(x1, a_s.T)
    Q1 = _dot(x1, a_d.T)
    xar = _dot(x1, a_r.T)
    Pn = P1 + xar
    Qn = Q1 + xar
    Rf = _dot(rel1, a_r.T)
    p1_ref[...] = P1
    pn_ref[...] = Pn
    q1_ref[...] = Q1
    qn_ref[...] = Qn
    rf_ref[...] = Rf
    p1s_ref[...] = _dot(P1, a2.T)
    q1s_ref[...] = _dot(Q1, a2.T)
    pns_ref[...] = _dot(Pn, a2.T)
    qns_ref[...] = _dot(Qn, a2.T)
    rrf_ref[...] = _dot(Rf, a2.T)


def _tc_mid(br, pcat, feats, rs0, rs1, rel_p, w, a_out, a2_out):
    NP = pcat.shape[0]
    RPR = rel_p.shape[0]
    c0 = lambda i: (0, 0)
    nblk = _blk((br, 128), lambda i: (i, 0))
    nsblk = _blk((br, 1), lambda i: (i, 0))
    rblk = _blk((RPR, 128), c0)
    rsblk = _blk((RPR, 1), c0)
    out_shape = (
        jax.ShapeDtypeStruct((RPR, 128), F32),
        jax.ShapeDtypeStruct((NP, 128), F32),
        jax.ShapeDtypeStruct((NP, 128), F32),
        jax.ShapeDtypeStruct((NP, 128), F32),
        jax.ShapeDtypeStruct((NP, 128), F32),
        jax.ShapeDtypeStruct((RPR, 128), F32),
        jax.ShapeDtypeStruct((NP, 1), F32),
        jax.ShapeDtypeStruct((NP, 1), F32),
        jax.ShapeDtypeStruct((NP, 1), F32),
        jax.ShapeDtypeStruct((NP, 1), F32),
        jax.ShapeDtypeStruct((RPR, 1), F32),
    )
    return pl.pallas_call(
        _tc_mid_body,
        grid=(NP // br,),
        in_specs=[nblk, nblk, nsblk, nsblk, rblk, _blk((128, 128), c0),
                  _blk((128, 384), c0), _blk((1, 128), c0)],
        out_specs=(rblk, nblk, nblk, nblk, nblk, rblk,
                   nsblk, nsblk, nsblk, nsblk, rsblk),
        out_shape=out_shape)(pcat, feats, rs0, rs1, rel_p, w, a_out, a2_out)


def _tc_post_body(p1_ref, pn_ref, feats_ref, rs1_ref, rsn_ref, out_ref):
    rs1 = rs1_ref[...]
    rsn = rsn_ref[...]
    rst = rs1 + rsn
    num = p1_ref[...] * rs1 + pn_ref[...] * rsn + feats_ref[...]
    hp = num / jnp.where(rst == 0.0, 1e-12, rst)
    out_ref[...] = _elu(hp)


def _tc_post(br, p1, pn, feats, rs1, rsn):
    NP = p1.shape[0]
    nblk = _blk((br, 128), lambda i: (i, 0))
    nsblk = _blk((br, 1), lambda i: (i, 0))
    return pl.pallas_call(
        _tc_post_body,
        grid=(NP // br,),
        in_specs=[nblk, nblk, nblk, nsblk, nsblk],
        out_specs=nblk,
        out_shape=jax.ShapeDtypeStruct((NP, 128), F32))(p1, pn, feats,
                                                        rs1, rsn)


def _sc_edge_kernel(np_rows, rp_rows, n1c, n2c, heads2):
    k1 = n1c // NSUB
    k2 = n2c // NSUB
    half = np_rows // NCORE
    drow = half
    rs_base = half + 16
    npo = rs_base + 2 * RSP
    npo = ((npo + 8 * NSUB - 1) // (8 * NSUB)) * (8 * NSUB)
    rps = npo // NSUB

    mesh = plsc.VectorSubcoreMesh(core_axis_name="c", subcore_axis_name="s")

    def body(src1_h, dst1_h, r01_h, srcn_h, dstn_h, r0n_h, r1n_h,
             qtab1_h, qtabn_h, rtab_h, pa_h, qa_h, pb_h, qb_h, rra_h, rrb_h,
             out_h,
             pa_v, qa_v, pb_v, qb_v, rra_v, rrb_v,
             si, di, r0i, r1i, si2, qrows, r0rows, r1rows, outbuf,
             rs0_v, rs1_v, idb, wa_v, wb_v, acc, sem):
        cid = lax.axis_index("c")
        sid = lax.axis_index("s")
        base = cid * half

        pltpu.sync_copy(pa_h, pa_v)
        pltpu.sync_copy(qa_h, qa_v)
        pltpu.sync_copy(pb_h, pb_v)
        pltpu.sync_copy(qb_h, qb_v)
        pltpu.sync_copy(rra_h, rra_v)
        pltpu.sync_copy(rrb_h, rrb_v)

        zeros16 = jnp.zeros((16,), F32)

        @pl.loop(0, RSP)
        def _(r):
            for k in range(8):
                rs0_v[r, pl.ds(k * 16, 16)] = zeros16
                rs1_v[r, pl.ds(k * 16, 16)] = zeros16

        @pl.loop(0, C)
        def _(r):
            for k in range(8):
                outbuf[r, pl.ds(k * 16, 16)] = zeros16

        nfull = rps // C
        rem = rps - nfull * C
        for b in range(nfull):
            pltpu.sync_copy(outbuf, acc.at[pl.ds(sid * rps + b * C, C)])
        if rem:
            pltpu.sync_copy(outbuf.at[pl.ds(0, rem)],
                            acc.at[pl.ds(sid * rps + nfull * C, rem)])
        plsc.subcore_barrier()

        lanes = lax.iota(jnp.int32, 16)

        def weights(k, pv, qv, rrv, two_rel):
            iv = si[0, pl.ds(k * 16, 16)]
            jv = di[0, pl.ds(k * 16, 16)]
            rv = r0i[0, pl.ds(k * 16, 16)]
            s = (plsc.load_gather(pv, [iv])
                 + plsc.load_gather(qv, [jv])
                 + plsc.load_gather(rrv, [rv]))
            if two_rel:
                r2v = r1i[0, pl.ds(k * 16, 16)]
                s = s + plsc.load_gather(rrv, [r2v])
            return jnp.exp(-_lrelu(s))

        def rs_update(rs_v, loc, valid, w):
            locc = jnp.where(valid, loc, 0)
            ridx = lax.shift_right_logical(locc, 7)
            cidx = lax.bitwise_and(locc, 127)
            for l in range(16):
                plsc.addupdate_scatter(rs_v, [ridx, cidx], w,
                                       mask=(lanes == l) & valid)

        def do_chunk(srcs, dsts, r0s, r1s, qtab, onehop, g):
            pltpu.sync_copy(srcs.at[pl.ds(g, 1)], si)
            pltpu.sync_copy(dsts.at[pl.ds(g, 1)], di)
            pltpu.sync_copy(r0s.at[pl.ds(g, 1)], r0i)
            two_rel = r1s is not None
            if two_rel:
                pltpu.sync_copy(r1s.at[pl.ds(g, 1)], r1i)
            pltpu.async_copy(qtab.at[di.at[0]], qrows, sem).wait()
            pltpu.async_copy(rtab_h.at[r0i.at[0]], r0rows, sem).wait()
            if two_rel:
                pltpu.async_copy(rtab_h.at[r1i.at[0]], r1rows, sem).wait()

            @pl.loop(0, C // 16)
            def _(k):
                iv = si[0, pl.ds(k * 16, 16)]
                loc = iv - base
                valid = (loc >= 0) & (loc < half)
                si2[0, pl.ds(k * 16, 16)] = jnp.where(valid, loc, drow)
                if heads2:
                    w0 = weights(k, pa_v, qa_v, rra_v, two_rel)
                    w1 = weights(k, pb_v, qb_v, rrb_v, two_rel)
                    rs_update(rs0_v, loc, valid, w0)
                    rs_update(rs1_v, loc, valid, w1)
                    wa_v[pl.ds(k * 16, 16)] = w0
                    wb_v[pl.ds(k * 16, 16)] = w1
                elif onehop:
                    w0 = weights(k, pa_v, qa_v, rra_v, two_rel)
                    rs_update(rs0_v, loc, valid, w0)
                    wa_v[pl.ds(k * 16, 16)] = w0
                else:
                    w1 = weights(k, pb_v, qb_v, rra_v, two_rel)
                    rs_update(rs1_v, loc, valid, w1)
                    wb_v[pl.ds(k * 16, 16)] = w1

            @pl.loop(0, C)
            def _(ci):
                idxc = jnp.full((16,), ci, jnp.int32)
                if heads2:
                    w0 = plsc.load_gather(wa_v, [idxc])
                    w1 = plsc.load_gather(wb_v, [idxc])
                elif onehop:
                    w0 = w1 = plsc.load_gather(wa_v, [idxc])
                else:
                    w0 = w1 = plsc.load_gather(wb_v, [idxc])
                for k in range(8):
                    v = (qrows[ci, pl.ds(k * 16, 16)]
                         + r0rows[ci, pl.ds(k * 16, 16)])
                    if two_rel:
                        v = v + r1rows[ci, pl.ds(k * 16, 16)]
                    outbuf[ci, pl.ds(k * 16, 16)] = (w0 if k < 4 else w1) * v

            pltpu.sync_copy(outbuf, acc.at[si2.at[0]], add=True)

        if k1:
            @pl.loop(0, k1)
            def _(t):
                do_chunk(src1_h, dst1_h, r01_h, None, qtab1_h, True,
                         sid * k1 + t)

        if k2:
            @pl.loop(0, k2)
            def _(t):
                do_chunk(srcn_h, dstn_h, r0n_h, r1n_h, qtabn_h, False,
                         sid * k2 + t)

        for slot, rs_v in ((0, rs0_v), (1, rs1_v)):
            rbase = rs_base + slot * RSP

            @pl.loop(0, RSP // 16)
            def _(k):
                idb[0, pl.ds(k * 16, 16)] = lanes + (rbase + k * 16)
            pltpu.sync_copy(rs_v, acc.at[idb.at[0]], add=True)

        plsc.subcore_barrier()
        pltpu.sync_copy(acc.at[pl.ds(sid * rps, rps)],
                        out_h.at[cid, pl.ds(sid * rps, rps)])

    cp = pltpu.CompilerParams()
    if "needs_layout_passes" in pltpu.CompilerParams.__dataclass_fields__:
        cp = dataclasses.replace(cp, needs_layout_passes=False)

    kern = pl.kernel(
        body,
        mesh=mesh,
        compiler_params=cp,
        out_type=jax.ShapeDtypeStruct((NCORE, npo, 128), F32),
        scratch_types=[
            pltpu.VMEM((np_rows,), F32),
            pltpu.VMEM((np_rows,), F32),
            pltpu.VMEM((np_rows,), F32),
            pltpu.VMEM((np_rows,), F32),
            pltpu.VMEM((rp_rows,), F32),
            pltpu.VMEM((rp_rows,), F32),
            pltpu.VMEM((1, C), jnp.int32),
            pltpu.VMEM((1, C), jnp.int32),
            pltpu.VMEM((1, C), jnp.int32),
            pltpu.VMEM((1, C), jnp.int32),
            pltpu.VMEM((1, C), jnp.int32),
            pltpu.VMEM((C, 128), F32),
            pltpu.VMEM((C, 128), F32),
            pltpu.VMEM((C, 128), F32),
            pltpu.VMEM((C, 128), F32),
            pltpu.VMEM((RSP, 128), F32),
            pltpu.VMEM((RSP, 128), F32),
            pltpu.VMEM((1, RSP), jnp.int32),
            pltpu.VMEM((C,), F32),
            pltpu.VMEM((C,), F32),
            pltpu.VMEM_SHARED((npo, 128), F32),
            pltpu.SemaphoreType.DMA,
        ],
    )
    return kern, npo


def _chunked(v, length, fill):
    return jnp.pad(v, (0, length - v.shape[0]),
                   constant_values=fill).reshape(length // C, C)


def kernel(Corpus_, batch_inputs, entity_embeddings, relation_embeddings,
           entity_list, relation_type, entity_list_nhop, relation_type_nhop,
           W, a0, a20, a1, a21, a_out, a2_out):
    N = entity_embeddings.shape[0]
    R = relation_embeddings.shape[0]
    E1 = entity_list.shape[1]
    E2 = entity_list_nhop.shape[1]

    NP = ((N + 1 + 255) // 256) * 256
    RP = ((R + 1 + 7) // 8) * 8
    half = NP // NCORE
    EC = NSUB * C
    E1P = ((E1 + EC - 1) // EC) * EC
    E2P = ((E2 + EC - 1) // EC) * EC

    src1 = _chunked(entity_list[0], E1P, N)
    dst1 = _chunked(entity_list[1], E1P, N)
    rel1i = _chunked(relation_type, E1P, R)
    srcn = _chunked(entity_list_nhop[0], E2P, N)
    dstn = _chunked(entity_list_nhop[1], E2P, N)
    r0n = _chunked(relation_type_nhop[:, 0], E2P, R)
    r1n = _chunked(relation_type_nhop[:, 1], E2P, R)

    x_p = jnp.pad(entity_embeddings, ((0, NP - N), (0, 0)))
    rel_p = jnp.pad(relation_embeddings, ((0, RP - R), (0, 0)))

    def col(v):
        return v[:, 0]

    BR = 1280

    def unpack(acc, npo):
        feats = acc[:, :half, :].reshape(NP, 128)
        rsb = half + 16
        rs = [acc[:, rsb + s * RSP:rsb + (s + 1) * RSP, :]
              .reshape(2, RSP * 128)[:, :half].reshape(NP, 1) for s in (0, 1)]
        return feats, rs[0], rs[1]

    (qcat, rcat, pcat, p0, q0, p1h, q1h, rr0, rr1) = _tc_pre(
        BR, x_p, rel_p, a0, a20, a1, a21)

    sc1, npo1 = _sc_edge_kernel(NP, RP, E1P // C, E2P // C, True)
    acc01 = sc1(src1, dst1, rel1i, srcn, dstn, r0n, r1n,
                qcat, qcat, rcat,
                col(p0), col(q0), col(p1h), col(q1h), col(rr0), col(rr1))
    feats01, rsh0, rsh1 = unpack(acc01, npo1)

    (rel1, P1, Pn, Q1, Qn, Rf, p1s, q1s, pns, qns, rrf) = _tc_mid(
        BR, pcat, feats01, rsh0, rsh1, rel_p, W, a_out, a2_out)

    sc2, npo2 = _sc_edge_kernel(NP, RP, E1P // C, E2P // C, False)
    accf = sc2(src1, dst1, rel1i, srcn, dstn, r0n, r1n,
               Q1, Qn, Rf,
               col(p1s), col(q1s), col(pns), col(qns), col(rrf), col(rrf))
    featsf, rsf1, rsfn = unpack(accf, npo2)

    out_entity_final = _tc_post(BR, P1, Pn, featsf, rsf1, rsfn)
    return (out_entity_final[:N], rel1[:R])


# speedup vs baseline: 2.2257x; 1.3021x over previous
import dataclasses
import functools

import jax
import jax.numpy as jnp
from jax import lax
from jax.experimental import pallas as pl
from jax.experimental.pallas import tpu as pltpu
from jax.experimental.pallas import tpu_sc as plsc

ALPHA = 0.2
C = 64
NCORE = 2
NSUB = 16
RSP = 48
F32 = jnp.float32


def _lrelu(x):
    return jnp.where(x > 0, x, ALPHA * x)


def _elu(x):
    return jnp.where(x > 0, x, jnp.exp(x) - 1.0)


def _dot(a, b):
    return jax.lax.dot(a, b, precision=jax.lax.Precision.HIGHEST,
                       preferred_element_type=F32)


def _blk(shape, index_map):
    return pl.BlockSpec(shape, index_map)


def _tc_pre_body(x_ref, rel_ref, a0_ref, a20_ref, a1_ref, a21_ref,
                 qcat_ref, rcat_ref, pcat_ref,
                 p0_ref, q0_ref, p1_ref, q1_ref, rr0_ref, rr1_ref):
    x = x_ref[...]
    rel = rel_ref[...]
    for h, (a_ref, a2_ref) in enumerate(((a0_ref, a20_ref), (a1_ref, a21_ref))):
        a = a_ref[...]
        a_s = a[:, 0:128]
        a_d = a[:, 128:256]
        a_r = a[:, 256:384]
        a2 = a2_ref[...]
        P = _dot(x, (a_s + a_r).T)
        Q = _dot(x, (a_d + a_r).T)
        Rv = _dot(rel, a_r.T)
        sl = slice(h * 64, (h + 1) * 64)
        pcat_ref[:, sl] = P
        qcat_ref[:, sl] = Q
        rcat_ref[:, sl] = Rv
        (p0_ref if h == 0 else p1_ref)[...] = _dot(P, a2.T)
        (q0_ref if h == 0 else q1_ref)[...] = _dot(Q, a2.T)
        (rr0_ref if h == 0 else rr1_ref)[...] = _dot(Rv, a2.T)


def _tc_pre(br, x_p, rel_p, a0, a20, a1, a21):
    NP = x_p.shape[0]
    RPR = rel_p.shape[0]
    c0 = lambda i: (0, 0)
    nblk = _blk((br, 128), lambda i: (i, 0))
    nsblk = _blk((br, 1), lambda i: (i, 0))
    rblk = _blk((RPR, 128), c0)
    rsblk = _blk((RPR, 1), c0)
    out_shape = (
        jax.ShapeDtypeStruct((NP, 128), F32),
        jax.ShapeDtypeStruct((RPR, 128), F32),
        jax.ShapeDtypeStruct((NP, 128), F32),
        jax.ShapeDtypeStruct((NP, 1), F32),
        jax.ShapeDtypeStruct((NP, 1), F32),
        jax.ShapeDtypeStruct((NP, 1), F32),
        jax.ShapeDtypeStruct((NP, 1), F32),
        jax.ShapeDtypeStruct((RPR, 1), F32),
        jax.ShapeDtypeStruct((RPR, 1), F32),
    )
    return pl.pallas_call(
        _tc_pre_body,
        grid=(NP // br,),
        in_specs=[nblk, rblk, _blk((64, 384), c0), _blk((1, 64), c0),
                  _blk((64, 384), c0), _blk((1, 64), c0)],
        out_specs=(nblk, rblk, nblk, nsblk, nsblk, nsblk, nsblk,
                   rsblk, rsblk),
        out_shape=out_shape)(x_p, rel_p, a0, a20, a1, a21)


def _tc_mid_body(pcat_ref, feats_ref, rs0_ref, rs1_ref, rel_ref, w_ref, a_ref,
                 a2_ref, rel1_ref, p1_ref, pn_ref, q1_ref, qn_ref, rf_ref,
                 p1s_ref, q1s_ref, pns_ref, qns_ref, rrf_ref):
    heads = []
    for h, rs_ref in enumerate((rs0_ref, rs1_ref)):
        rs = rs_ref[...]
        P = pcat_ref[:, h * 64:(h + 1) * 64]
        hacc = feats_ref[:, h * 64:(h + 1) * 64]
        hp = jnp.where(rs > 0, P + hacc / jnp.where(rs > 0, rs, 1.0), 0.0)
        heads.append(_elu(hp))
    x1 = jnp.concatenate(heads, axis=1)
    rel1 = _dot(rel_ref[...], w_ref[...])
    rel1_ref[...] = rel1
    a = a_ref[...]
    a_s = a[:, 0:128]
    a_d = a[:, 128:256]
    a_r = a[:, 256:384]
    a2 = a2_ref[...]
    P1 = _dot(x1, a_s.T)
    Q1 = _dot(x1, a_d.T)
    xar = _dot(x1, a_r.T)
    Pn = P1 + xar
    Qn = Q1 + xar
    Rf = _dot(rel1, a_r.T)
    p1_ref[...] = P1
    pn_ref[...] = Pn
    q1_ref[...] = Q1
    qn_ref[...] = Qn
    rf_ref[...] = Rf
    p1s_ref[...] = _dot(P1, a2.T)
    q1s_ref[...] = _dot(Q1, a2.T)
    pns_ref[...] = _dot(Pn, a2.T)
    qns_ref[...] = _dot(Qn, a2.T)
    rrf_ref[...] = _dot(Rf, a2.T)


def _tc_mid(br, pcat, feats, rs0, rs1, rel_p, w, a_out, a2_out):
    NP = pcat.shape[0]
    RPR = rel_p.shape[0]
    c0 = lambda i: (0, 0)
    nblk = _blk((br, 128), lambda i: (i, 0))
    nsblk = _blk((br, 1), lambda i: (i, 0))
    rblk = _blk((RPR, 128), c0)
    rsblk = _blk((RPR, 1), c0)
    out_shape = (
        jax.ShapeDtypeStruct((RPR, 128), F32),
        jax.ShapeDtypeStruct((NP, 128), F32),
        jax.ShapeDtypeStruct((NP, 128), F32),
        jax.ShapeDtypeStruct((NP, 128), F32),
        jax.ShapeDtypeStruct((NP, 128), F32),
        jax.ShapeDtypeStruct((RPR, 128), F32),
        jax.ShapeDtypeStruct((NP, 1), F32),
        jax.ShapeDtypeStruct((NP, 1), F32),
        jax.ShapeDtypeStruct((NP, 1), F32),
        jax.ShapeDtypeStruct((NP, 1), F32),
        jax.ShapeDtypeStruct((RPR, 1), F32),
    )
    return pl.pallas_call(
        _tc_mid_body,
        grid=(NP // br,),
        in_specs=[nblk, nblk, nsblk, nsblk, rblk, _blk((128, 128), c0),
                  _blk((128, 384), c0), _blk((1, 128), c0)],
        out_specs=(rblk, nblk, nblk, nblk, nblk, rblk,
                   nsblk, nsblk, nsblk, nsblk, rsblk),
        out_shape=out_shape)(pcat, feats, rs0, rs1, rel_p, w, a_out, a2_out)


def _tc_post_body(p1_ref, pn_ref, feats_ref, rs1_ref, rsn_ref, out_ref):
    rs1 = rs1_ref[...]
    rsn = rsn_ref[...]
    rst = rs1 + rsn
    num = p1_ref[...] * rs1 + pn_ref[...] * rsn + feats_ref[...]
    hp = num / jnp.where(rst == 0.0, 1e-12, rst)
    out_ref[...] = _elu(hp)


def _tc_post(br, p1, pn, feats, rs1, rsn):
    NP = p1.shape[0]
    nblk = _blk((br, 128), lambda i: (i, 0))
    nsblk = _blk((br, 1), lambda i: (i, 0))
    return pl.pallas_call(
        _tc_post_body,
        grid=(NP // br,),
        in_specs=[nblk, nblk, nblk, nsblk, nsblk],
        out_specs=nblk,
        out_shape=jax.ShapeDtypeStruct((NP, 128), F32))(p1, pn, feats,
                                                        rs1, rsn)


def _sc_edge_kernel(np_rows, rp_rows, n1c, n2c, heads2):
    k1 = n1c // NSUB
    k2 = n2c // NSUB
    half = np_rows // NCORE
    drow = half
    rs_base = half + 16
    npo = rs_base + 2 * RSP
    npo = ((npo + 8 * NSUB - 1) // (8 * NSUB)) * (8 * NSUB)
    rps = npo // NSUB

    mesh = plsc.VectorSubcoreMesh(core_axis_name="c", subcore_axis_name="s")

    def body(idx1_h, idxn_h,
             qtab1_h, qtabn_h, rtab_h, pa_h, qa_h, pb_h, qb_h, rra_h, rrb_h,
             out_h,
             pa_v, qa_v, pb_v, qb_v, rra_v, rrb_v,
             idxb, si2, qrows, r0rows, r1rows, outbuf,
             rs0_v, rs1_v, idb, wa_v, wb_v, acc, sem):
        cid = lax.axis_index("c")
        sid = lax.axis_index("s")
        base = cid * half

        pltpu.sync_copy(pa_h, pa_v)
        pltpu.sync_copy(qa_h, qa_v)
        pltpu.sync_copy(pb_h, pb_v)
        pltpu.sync_copy(qb_h, qb_v)
        pltpu.sync_copy(rra_h, rra_v)
        pltpu.sync_copy(rrb_h, rrb_v)

        zeros16 = jnp.zeros((16,), F32)

        @pl.loop(0, RSP)
        def _(r):
            for k in range(8):
                rs0_v[r, pl.ds(k * 16, 16)] = zeros16
                rs1_v[r, pl.ds(k * 16, 16)] = zeros16

        @pl.loop(0, C)
        def _(r):
            for k in range(8):
                outbuf[r, pl.ds(k * 16, 16)] = zeros16

        nfull = rps // C
        rem = rps - nfull * C
        for b in range(nfull):
            pltpu.sync_copy(outbuf, acc.at[pl.ds(sid * rps + b * C, C)])
        if rem:
            pltpu.sync_copy(outbuf.at[pl.ds(0, rem)],
                            acc.at[pl.ds(sid * rps + nfull * C, rem)])
        plsc.subcore_barrier()

        lanes = lax.iota(jnp.int32, 16)

        def weights(k, pv, qv, rrv, two_rel):
            iv = idxb[0, pl.ds(k * 16, 16)]
            jv = idxb[1, pl.ds(k * 16, 16)]
            rv = idxb[2, pl.ds(k * 16, 16)]
            s = (plsc.load_gather(pv, [iv])
                 + plsc.load_gather(qv, [jv])
                 + plsc.load_gather(rrv, [rv]))
            if two_rel:
                r2v = idxb[3, pl.ds(k * 16, 16)]
                s = s + plsc.load_gather(rrv, [r2v])
            return jnp.exp(-_lrelu(s))

        def rs_update(rs_v, loc, valid, w):
            locc = jnp.where(valid, loc, 0)
            ridx = lax.shift_right_logical(locc, 7)
            cidx = lax.bitwise_and(locc, 127)
            for l in range(16):
                plsc.addupdate_scatter(rs_v, [ridx, cidx], w,
                                       mask=(lanes == l) & valid)

        def do_chunk(idxs, two_rel, qtab, onehop, g):
            pltpu.sync_copy(idxs.at[g], idxb)
            h1 = pltpu.async_copy(qtab.at[idxb.at[1]], qrows, sem)
            h2 = pltpu.async_copy(rtab_h.at[idxb.at[2]], r0rows, sem)
            h3 = (pltpu.async_copy(rtab_h.at[idxb.at[3]], r1rows, sem)
                  if two_rel else None)
            h1.wait()
            h2.wait()
            if two_rel:
                h3.wait()

            @pl.loop(0, C // 16)
            def _(k):
                iv = idxb[0, pl.ds(k * 16, 16)]
                loc = iv - base
                valid = (loc >= 0) & (loc < half)
                si2[0, pl.ds(k * 16, 16)] = jnp.where(valid, loc, drow)
                if heads2:
                    w0 = weights(k, pa_v, qa_v, rra_v, two_rel)
                    w1 = weights(k, pb_v, qb_v, rrb_v, two_rel)
                    rs_update(rs0_v, loc, valid, w0)
                    rs_update(rs1_v, loc, valid, w1)
                    wa_v[pl.ds(k * 16, 16)] = w0
                    wb_v[pl.ds(k * 16, 16)] = w1
                elif onehop:
                    w0 = weights(k, pa_v, qa_v, rra_v, two_rel)
                    rs_update(rs0_v, loc, valid, w0)
                    wa_v[pl.ds(k * 16, 16)] = w0
                else:
                    w1 = weights(k, pb_v, qb_v, rra_v, two_rel)
                    rs_update(rs1_v, loc, valid, w1)
                    wb_v[pl.ds(k * 16, 16)] = w1

            @pl.loop(0, C)
            def _(ci):
                idxc = jnp.full((16,), ci, jnp.int32)
                if heads2:
                    w0 = plsc.load_gather(wa_v, [idxc])
                    w1 = plsc.load_gather(wb_v, [idxc])
                elif onehop:
                    w0 = w1 = plsc.load_gather(wa_v, [idxc])
                else:
                    w0 = w1 = plsc.load_gather(wb_v, [idxc])
                for k in range(8):
                    v = (qrows[ci, pl.ds(k * 16, 16)]
                         + r0rows[ci, pl.ds(k * 16, 16)])
                    if two_rel:
                        v = v + r1rows[ci, pl.ds(k * 16, 16)]
                    outbuf[ci, pl.ds(k * 16, 16)] = (w0 if k < 4 else w1) * v

            pltpu.sync_copy(outbuf, acc.at[si2.at[0]], add=True)

        if k1:
            @pl.loop(0, k1)
            def _(t):
                do_chunk(idx1_h, False, qtab1_h, True, sid * k1 + t)

        if k2:
            @pl.loop(0, k2)
            def _(t):
                do_chunk(idxn_h, True, qtabn_h, False, sid * k2 + t)

        for slot, rs_v in ((0, rs0_v), (1, rs1_v)):
            rbase = rs_base + slot * RSP

            @pl.loop(0, RSP // 16)
            def _(k):
                idb[0, pl.ds(k * 16, 16)] = lanes + (rbase + k * 16)
            pltpu.sync_copy(rs_v, acc.at[idb.at[0]], add=True)

        plsc.subcore_barrier()
        pltpu.sync_copy(acc.at[pl.ds(sid * rps, rps)],
                        out_h.at[cid, pl.ds(sid * rps, rps)])

    cp = pltpu.CompilerParams()
    if "needs_layout_passes" in pltpu.CompilerParams.__dataclass_fields__:
        cp = dataclasses.replace(cp, needs_layout_passes=False)

    kern = pl.kernel(
        body,
        mesh=mesh,
        compiler_params=cp,
        out_type=jax.ShapeDtypeStruct((NCORE, npo, 128), F32),
        scratch_types=[
            pltpu.VMEM((np_rows,), F32),
            pltpu.VMEM((np_rows,), F32),
            pltpu.VMEM((np_rows,), F32),
            pltpu.VMEM((np_rows,), F32),
            pltpu.VMEM((rp_rows,), F32),
            pltpu.VMEM((rp_rows,), F32),
            pltpu.VMEM((4, C), jnp.int32),
            pltpu.VMEM((1, C), jnp.int32),
            pltpu.VMEM((C, 128), F32),
            pltpu.VMEM((C, 128), F32),
            pltpu.VMEM((C, 128), F32),
            pltpu.VMEM((C, 128), F32),
            pltpu.VMEM((RSP, 128), F32),
            pltpu.VMEM((RSP, 128), F32),
            pltpu.VMEM((1, RSP), jnp.int32),
            pltpu.VMEM((C,), F32),
            pltpu.VMEM((C,), F32),
            pltpu.VMEM_SHARED((npo, 128), F32),
            pltpu.SemaphoreType.DMA,
        ],
    )
    return kern, npo


def _chunked(v, length, fill):
    return jnp.pad(v, (0, length - v.shape[0]),
                   constant_values=fill).reshape(length // C, C)


def kernel(Corpus_, batch_inputs, entity_embeddings, relation_embeddings,
           entity_list, relation_type, entity_list_nhop, relation_type_nhop,
           W, a0, a20, a1, a21, a_out, a2_out):
    N = entity_embeddings.shape[0]
    R = relation_embeddings.shape[0]
    E1 = entity_list.shape[1]
    E2 = entity_list_nhop.shape[1]

    NP = ((N + 1 + 255) // 256) * 256
    RP = ((R + 1 + 7) // 8) * 8
    half = NP // NCORE
    EC = NSUB * C
    E1P = ((E1 + EC - 1) // EC) * EC
    E2P = ((E2 + EC - 1) // EC) * EC

    src1 = _chunked(entity_list[0], E1P, N)
    dst1 = _chunked(entity_list[1], E1P, N)
    rel1i = _chunked(relation_type, E1P, R)
    idx1 = jnp.stack([src1, dst1, rel1i, rel1i], axis=1)
    srcn = _chunked(entity_list_nhop[0], E2P, N)
    dstn = _chunked(entity_list_nhop[1], E2P, N)
    r0n = _chunked(relation_type_nhop[:, 0], E2P, R)
    r1n = _chunked(relation_type_nhop[:, 1], E2P, R)
    idxn = jnp.stack([srcn, dstn, r0n, r1n], axis=1)

    x_p = jnp.pad(entity_embeddings, ((0, NP - N), (0, 0)))
    rel_p = jnp.pad(relation_embeddings, ((0, RP - R), (0, 0)))

    def col(v):
        return v[:, 0]

    BR = 1280

    def unpack(acc, npo):
        feats = acc[:, :half, :].reshape(NP, 128)
        rsb = half + 16
        rs = [acc[:, rsb + s * RSP:rsb + (s + 1) * RSP, :]
              .reshape(2, RSP * 128)[:, :half].reshape(NP, 1) for s in (0, 1)]
        return feats, rs[0], rs[1]

    (qcat, rcat, pcat, p0, q0, p1h, q1h, rr0, rr1) = _tc_pre(
        BR, x_p, rel_p, a0, a20, a1, a21)

    sc1, npo1 = _sc_edge_kernel(NP, RP, E1P // C, E2P // C, True)
    acc01 = sc1(idx1, idxn,
                qcat, qcat, rcat,
                col(p0), col(q0), col(p1h), col(q1h), col(rr0), col(rr1))
    feats01, rsh0, rsh1 = unpack(acc01, npo1)

    (rel1, P1, Pn, Q1, Qn, Rf, p1s, q1s, pns, qns, rrf) = _tc_mid(
        BR, pcat, feats01, rsh0, rsh1, rel_p, W, a_out, a2_out)

    sc2, npo2 = _sc_edge_kernel(NP, RP, E1P // C, E2P // C, False)
    accf = sc2(idx1, idxn,
               Q1, Qn, Rf,
               col(p1s), col(q1s), col(pns), col(qns), col(rrf), col(rrf))
    featsf, rsf1, rsfn = unpack(accf, npo2)

    out_entity_final = _tc_post(BR, P1, Pn, featsf, rsf1, rsfn)
    return (out_entity_final[:N], rel1[:R])


# speedup vs baseline: 2.6711x; 1.2001x over previous
import dataclasses
import functools

import jax
import jax.numpy as jnp
from jax import lax
from jax.experimental import pallas as pl
from jax.experimental.pallas import tpu as pltpu
from jax.experimental.pallas import tpu_sc as plsc

ALPHA = 0.2
C = 32
NCORE = 2
NSUB = 16
RSP = 48
F32 = jnp.float32


def _lrelu(x):
    return jnp.where(x > 0, x, ALPHA * x)


def _elu(x):
    return jnp.where(x > 0, x, jnp.exp(x) - 1.0)


def _dot(a, b):
    return jax.lax.dot(a, b, precision=jax.lax.Precision.HIGHEST,
                       preferred_element_type=F32)


def _blk(shape, index_map):
    return pl.BlockSpec(shape, index_map)


def _tc_pre_body(x_ref, rel_ref, a0_ref, a20_ref, a1_ref, a21_ref,
                 qcat_ref, rcat_ref, pcat_ref,
                 p0_ref, q0_ref, p1_ref, q1_ref, rr0_ref, rr1_ref):
    x = x_ref[...]
    rel = rel_ref[...]
    for h, (a_ref, a2_ref) in enumerate(((a0_ref, a20_ref), (a1_ref, a21_ref))):
        a = a_ref[...]
        a_s = a[:, 0:128]
        a_d = a[:, 128:256]
        a_r = a[:, 256:384]
        a2 = a2_ref[...]
        P = _dot(x, (a_s + a_r).T)
        Q = _dot(x, (a_d + a_r).T)
        Rv = _dot(rel, a_r.T)
        sl = slice(h * 64, (h + 1) * 64)
        pcat_ref[:, sl] = P
        qcat_ref[:, sl] = Q
        rcat_ref[:, sl] = Rv
        (p0_ref if h == 0 else p1_ref)[...] = _dot(P, a2.T)
        (q0_ref if h == 0 else q1_ref)[...] = _dot(Q, a2.T)
        (rr0_ref if h == 0 else rr1_ref)[...] = _dot(Rv, a2.T)


def _tc_pre(br, x_p, rel_p, a0, a20, a1, a21):
    NP = x_p.shape[0]
    RPR = rel_p.shape[0]
    c0 = lambda i: (0, 0)
    nblk = _blk((br, 128), lambda i: (i, 0))
    nsblk = _blk((br, 1), lambda i: (i, 0))
    rblk = _blk((RPR, 128), c0)
    rsblk = _blk((RPR, 1), c0)
    out_shape = (
        jax.ShapeDtypeStruct((NP, 128), F32),
        jax.ShapeDtypeStruct((RPR, 128), F32),
        jax.ShapeDtypeStruct((NP, 128), F32),
        jax.ShapeDtypeStruct((NP, 1), F32),
        jax.ShapeDtypeStruct((NP, 1), F32),
        jax.ShapeDtypeStruct((NP, 1), F32),
        jax.ShapeDtypeStruct((NP, 1), F32),
        jax.ShapeDtypeStruct((RPR, 1), F32),
        jax.ShapeDtypeStruct((RPR, 1), F32),
    )
    return pl.pallas_call(
        _tc_pre_body,
        grid=(NP // br,),
        in_specs=[nblk, rblk, _blk((64, 384), c0), _blk((1, 64), c0),
                  _blk((64, 384), c0), _blk((1, 64), c0)],
        out_specs=(nblk, rblk, nblk, nsblk, nsblk, nsblk, nsblk,
                   rsblk, rsblk),
        out_shape=out_shape)(x_p, rel_p, a0, a20, a1, a21)


def _tc_mid_body(pcat_ref, feats_ref, rs0_ref, rs1_ref, rel_ref, w_ref, a_ref,
                 a2_ref, rel1_ref, p1_ref, pn_ref, q1_ref, qn_ref, rf_ref,
                 p1s_ref, q1s_ref, pns_ref, qns_ref, rrf_ref):
    heads = []
    for h, rs_ref in enumerate((rs0_ref, rs1_ref)):
        rs = rs_ref[...]
        P = pcat_ref[:, h * 64:(h + 1) * 64]
        hacc = feats_ref[:, h * 64:(h + 1) * 64]
        hp = jnp.where(rs > 0, P + hacc / jnp.where(rs > 0, rs, 1.0), 0.0)
        heads.append(_elu(hp))
    x1 = jnp.concatenate(heads, axis=1)
    rel1 = _dot(rel_ref[...], w_ref[...])
    rel1_ref[...] = rel1
    a = a_ref[...]
    a_s = a[:, 0:128]
    a_d = a[:, 128:256]
    a_r = a[:, 256:384]
    a2 = a2_ref[...]
    P1 = _dot(x1, a_s.T)
    Q1 = _dot(x1, a_d.T)
    xar = _dot(x1, a_r.T)
    Pn = P1 + xar
    Qn = Q1 + xar
    Rf = _dot(rel1, a_r.T)
    p1_ref[...] = P1
    pn_ref[...] = Pn
    q1_ref[...] = Q1
    qn_ref[...] = Qn
    rf_ref[...] = Rf
    p1s_ref[...] = _dot(P1, a2.T)
    q1s_ref[...] = _dot(Q1, a2.T)
    pns_ref[...] = _dot(Pn, a2.T)
    qns_ref[...] = _dot(Qn, a2.T)
    rrf_ref[...] = _dot(Rf, a2.T)


def _tc_mid(br, pcat, feats, rs0, rs1, rel_p, w, a_out, a2_out):
    NP = pcat.shape[0]
    RPR = rel_p.shape[0]
    c0 = lambda i: (0, 0)
    nblk = _blk((br, 128), lambda i: (i, 0))
    nsblk = _blk((br, 1), lambda i: (i, 0))
    rblk = _blk((RPR, 128), c0)
    rsblk = _blk((RPR, 1), c0)
    out_shape = (
        jax.ShapeDtypeStruct((RPR, 128), F32),
        jax.ShapeDtypeStruct((NP, 128), F32),
        jax.ShapeDtypeStruct((NP, 128), F32),
        jax.ShapeDtypeStruct((NP, 128), F32),
        jax.ShapeDtypeStruct((NP, 128), F32),
        jax.ShapeDtypeStruct((RPR, 128), F32),
        jax.ShapeDtypeStruct((NP, 1), F32),
        jax.ShapeDtypeStruct((NP, 1), F32),
        jax.ShapeDtypeStruct((NP, 1), F32),
        jax.ShapeDtypeStruct((NP, 1), F32),
        jax.ShapeDtypeStruct((RPR, 1), F32),
    )
    return pl.pallas_call(
        _tc_mid_body,
        grid=(NP // br,),
        in_specs=[nblk, nblk, nsblk, nsblk, rblk, _blk((128, 128), c0),
                  _blk((128, 384), c0), _blk((1, 128), c0)],
        out_specs=(rblk, nblk, nblk, nblk, nblk, rblk,
                   nsblk, nsblk, nsblk, nsblk, rsblk),
        out_shape=out_shape)(pcat, feats, rs0, rs1, rel_p, w, a_out, a2_out)


def _tc_post_body(p1_ref, pn_ref, feats_ref, rs1_ref, rsn_ref, out_ref):
    rs1 = rs1_ref[...]
    rsn = rsn_ref[...]
    rst = rs1 + rsn
    num = p1_ref[...] * rs1 + pn_ref[...] * rsn + feats_ref[...]
    hp = num / jnp.where(rst == 0.0, 1e-12, rst)
    out_ref[...] = _elu(hp)


def _tc_post(br, p1, pn, feats, rs1, rsn):
    NP = p1.shape[0]
    nblk = _blk((br, 128), lambda i: (i, 0))
    nsblk = _blk((br, 1), lambda i: (i, 0))
    return pl.pallas_call(
        _tc_post_body,
        grid=(NP // br,),
        in_specs=[nblk, nblk, nblk, nsblk, nsblk],
        out_specs=nblk,
        out_shape=jax.ShapeDtypeStruct((NP, 128), F32))(p1, pn, feats,
                                                        rs1, rsn)


def _sc_edge_kernel(np_rows, rp_rows, n1c, n2c, heads2):
    k1 = n1c // NSUB
    k2 = n2c // NSUB
    half = np_rows // NCORE
    drow = half
    rs_base = half + 16
    npo = rs_base + 2 * RSP
    npo = ((npo + 8 * NSUB - 1) // (8 * NSUB)) * (8 * NSUB)
    rps = npo // NSUB

    mesh = plsc.VectorSubcoreMesh(core_axis_name="c", subcore_axis_name="s")

    def body(idx1_h, idxn_h,
             qtab1_h, qtabn_h, rtab_h, pa_h, qa_h, pb_h, qb_h, rra_h, rrb_h,
             out_h,
             pa_v, qa_v, pb_v, qb_v, rra_v, rrb_v,
             idxb0, idxb1, si2, qrows0, qrows1, r0rows0, r0rows1,
             r1rows0, r1rows1, outbuf,
             rs0_v, rs1_v, idb, wa_v, wb_v, acc, sem0, sem1, sem):
        cid = lax.axis_index("c")
        sid = lax.axis_index("s")
        base = cid * half

        pltpu.sync_copy(pa_h, pa_v)
        pltpu.sync_copy(qa_h, qa_v)
        pltpu.sync_copy(pb_h, pb_v)
        pltpu.sync_copy(qb_h, qb_v)
        pltpu.sync_copy(rra_h, rra_v)
        pltpu.sync_copy(rrb_h, rrb_v)

        zeros16 = jnp.zeros((16,), F32)

        @pl.loop(0, RSP)
        def _(r):
            for k in range(8):
                rs0_v[r, pl.ds(k * 16, 16)] = zeros16
                rs1_v[r, pl.ds(k * 16, 16)] = zeros16

        @pl.loop(0, C)
        def _(r):
            for k in range(8):
                outbuf[r, pl.ds(k * 16, 16)] = zeros16

        nfull = rps // C
        rem = rps - nfull * C
        for b in range(nfull):
            pltpu.sync_copy(outbuf, acc.at[pl.ds(sid * rps + b * C, C)])
        if rem:
            pltpu.sync_copy(outbuf.at[pl.ds(0, rem)],
                            acc.at[pl.ds(sid * rps + nfull * C, rem)])
        plsc.subcore_barrier()

        lanes = lax.iota(jnp.int32, 16)

        def rs_update(rs_v, loc, valid, w):
            locc = jnp.where(valid, loc, 0)
            ridx = lax.shift_right_logical(locc, 7)
            cidx = lax.bitwise_and(locc, 127)
            for l in range(16):
                plsc.addupdate_scatter(rs_v, [ridx, cidx], w,
                                       mask=(lanes == l) & valid)

        def prefetch(bufs, idxs, two_rel, qtab, g):
            idxb, qrows, r0rows, r1rows, sem = bufs
            pltpu.sync_copy(idxs.at[g], idxb)
            pltpu.async_copy(qtab.at[idxb.at[1]], qrows, sem)
            pltpu.async_copy(rtab_h.at[idxb.at[2]], r0rows, sem)
            if two_rel:
                pltpu.async_copy(rtab_h.at[idxb.at[3]], r1rows, sem)

        def process(bufs, two_rel, qtab, onehop):
            idxb, qrows, r0rows, r1rows, sem = bufs
            pltpu.make_async_copy(qtab.at[idxb.at[1]], qrows, sem).wait()
            pltpu.make_async_copy(rtab_h.at[idxb.at[2]], r0rows, sem).wait()
            if two_rel:
                pltpu.make_async_copy(rtab_h.at[idxb.at[3]], r1rows,
                                      sem).wait()

            @pl.loop(0, C // 16)
            def _(k):
                def weights(pv, qv, rrv):
                    iv = idxb[0, pl.ds(k * 16, 16)]
                    jv = idxb[1, pl.ds(k * 16, 16)]
                    rv = idxb[2, pl.ds(k * 16, 16)]
                    s = (plsc.load_gather(pv, [iv])
                         + plsc.load_gather(qv, [jv])
                         + plsc.load_gather(rrv, [rv]))
                    if two_rel:
                        r2v = idxb[3, pl.ds(k * 16, 16)]
                        s = s + plsc.load_gather(rrv, [r2v])
                    return jnp.exp(-_lrelu(s))

                iv = idxb[0, pl.ds(k * 16, 16)]
                loc = iv - base
                valid = (loc >= 0) & (loc < half)
                si2[0, pl.ds(k * 16, 16)] = jnp.where(valid, loc, drow)
                if heads2:
                    w0 = weights(pa_v, qa_v, rra_v)
                    w1 = weights(pb_v, qb_v, rrb_v)
                    rs_update(rs0_v, loc, valid, w0)
                    rs_update(rs1_v, loc, valid, w1)
                    wa_v[pl.ds(k * 16, 16)] = w0
                    wb_v[pl.ds(k * 16, 16)] = w1
                elif onehop:
                    w0 = weights(pa_v, qa_v, rra_v)
                    rs_update(rs0_v, loc, valid, w0)
                    wa_v[pl.ds(k * 16, 16)] = w0
                else:
                    w1 = weights(pb_v, qb_v, rra_v)
                    rs_update(rs1_v, loc, valid, w1)
                    wb_v[pl.ds(k * 16, 16)] = w1

            @pl.loop(0, C)
            def _(ci):
                idxc = jnp.full((16,), ci, jnp.int32)
                if heads2:
                    w0 = plsc.load_gather(wa_v, [idxc])
                    w1 = plsc.load_gather(wb_v, [idxc])
                elif onehop:
                    w0 = w1 = plsc.load_gather(wa_v, [idxc])
                else:
                    w0 = w1 = plsc.load_gather(wb_v, [idxc])
                for k in range(8):
                    v = (qrows[ci, pl.ds(k * 16, 16)]
                         + r0rows[ci, pl.ds(k * 16, 16)])
                    if two_rel:
                        v = v + r1rows[ci, pl.ds(k * 16, 16)]
                    outbuf[ci, pl.ds(k * 16, 16)] = (w0 if k < 4 else w1) * v

            pltpu.sync_copy(outbuf, acc.at[si2.at[0]], add=True)

        def sweep(idxs, nchunks, two_rel, qtab, onehop):
            bufs0 = (idxb0, qrows0, r0rows0, r1rows0, sem0)
            bufs1 = (idxb1, qrows1, r0rows1, r1rows1, sem1)
            g0 = sid * nchunks
            prefetch(bufs0, idxs, two_rel, qtab, g0)

            @pl.loop(0, (nchunks + 1) // 2)
            def _(th):
                t0 = 2 * th

                @pl.when(t0 + 1 < nchunks)
                def _():
                    prefetch(bufs1, idxs, two_rel, qtab, g0 + t0 + 1)
                process(bufs0, two_rel, qtab, onehop)

                @pl.when(t0 + 1 < nchunks)
                def _():
                    @pl.when(t0 + 2 < nchunks)
                    def _():
                        prefetch(bufs0, idxs, two_rel, qtab, g0 + t0 + 2)
                    process(bufs1, two_rel, qtab, onehop)

        if k1:
            sweep(idx1_h, k1, False, qtab1_h, True)
        if k2:
            sweep(idxn_h, k2, True, qtabn_h, False)

        for slot, rs_v in ((0, rs0_v), (1, rs1_v)):
            rbase = rs_base + slot * RSP
            for o in range(0, RSP, 16):
                idb[0, pl.ds(0, 16)] = lanes + (rbase + o)

                @pl.loop(0, 16)
                def _(r):
                    for k in range(8):
                        outbuf[r, pl.ds(k * 16, 16)] = \
                            rs_v[o + r, pl.ds(k * 16, 16)]
                pltpu.sync_copy(outbuf.at[pl.ds(0, 16)], acc.at[idb.at[0]],
                                add=True)

        plsc.subcore_barrier()
        pltpu.sync_copy(acc.at[pl.ds(sid * rps, rps)],
                        out_h.at[cid, pl.ds(sid * rps, rps)])

    cp = pltpu.CompilerParams()
    if "needs_layout_passes" in pltpu.CompilerParams.__dataclass_fields__:
        cp = dataclasses.replace(cp, needs_layout_passes=False)

    kern = pl.kernel(
        body,
        mesh=mesh,
        compiler_params=cp,
        out_type=jax.ShapeDtypeStruct((NCORE, npo, 128), F32),
        scratch_types=[
            pltpu.VMEM((np_rows,), F32),
            pltpu.VMEM((np_rows,), F32),
            pltpu.VMEM((np_rows,), F32),
            pltpu.VMEM((np_rows,), F32),
            pltpu.VMEM((rp_rows,), F32),
            pltpu.VMEM((rp_rows,), F32),
            pltpu.VMEM((4, C), jnp.int32),
            pltpu.VMEM((4, C), jnp.int32),
            pltpu.VMEM((1, C), jnp.int32),
            pltpu.VMEM((C, 128), F32),
            pltpu.VMEM((C, 128), F32),
            pltpu.VMEM((C, 128), F32),
            pltpu.VMEM((C, 128), F32),
            pltpu.VMEM((C, 128), F32),
            pltpu.VMEM((C, 128), F32),
            pltpu.VMEM((C, 128), F32),
            pltpu.VMEM((RSP, 128), F32),
            pltpu.VMEM((RSP, 128), F32),
            pltpu.VMEM((1, 16), jnp.int32),
            pltpu.VMEM((C,), F32),
            pltpu.VMEM((C,), F32),
            pltpu.VMEM_SHARED((npo, 128), F32),
            pltpu.SemaphoreType.DMA,
            pltpu.SemaphoreType.DMA,
            pltpu.SemaphoreType.DMA,
        ],
    )
    return kern, npo


def _chunked(v, length, fill):
    return jnp.pad(v, (0, length - v.shape[0]),
                   constant_values=fill).reshape(length // C, C)


def kernel(Corpus_, batch_inputs, entity_embeddings, relation_embeddings,
           entity_list, relation_type, entity_list_nhop, relation_type_nhop,
           W, a0, a20, a1, a21, a_out, a2_out):
    N = entity_embeddings.shape[0]
    R = relation_embeddings.shape[0]
    E1 = entity_list.shape[1]
    E2 = entity_list_nhop.shape[1]

    NP = ((N + 1 + 255) // 256) * 256
    RP = ((R + 1 + 7) // 8) * 8
    half = NP // NCORE
    EC = NSUB * C
    E1P = ((E1 + EC - 1) // EC) * EC
    E2P = ((E2 + EC - 1) // EC) * EC

    src1 = _chunked(entity_list[0], E1P, N)
    dst1 = _chunked(entity_list[1], E1P, N)
    rel1i = _chunked(relation_type, E1P, R)
    idx1 = jnp.stack([src1, dst1, rel1i, rel1i], axis=1)
    srcn = _chunked(entity_list_nhop[0], E2P, N)
    dstn = _chunked(entity_list_nhop[1], E2P, N)
    r0n = _chunked(relation_type_nhop[:, 0], E2P, R)
    r1n = _chunked(relation_type_nhop[:, 1], E2P, R)
    idxn = jnp.stack([srcn, dstn, r0n, r1n], axis=1)

    x_p = jnp.pad(entity_embeddings, ((0, NP - N), (0, 0)))
    rel_p = jnp.pad(relation_embeddings, ((0, RP - R), (0, 0)))

    def col(v):
        return v[:, 0]

    BR = 1280

    def unpack(acc, npo):
        feats = acc[:, :half, :].reshape(NP, 128)
        rsb = half + 16
        rs = [acc[:, rsb + s * RSP:rsb + (s + 1) * RSP, :]
              .reshape(2, RSP * 128)[:, :half].reshape(NP, 1) for s in (0, 1)]
        return feats, rs[0], rs[1]

    (qcat, rcat, pcat, p0, q0, p1h, q1h, rr0, rr1) = _tc_pre(
        BR, x_p, rel_p, a0, a20, a1, a21)

    sc1, npo1 = _sc_edge_kernel(NP, RP, E1P // C, E2P // C, True)
    acc01 = sc1(idx1, idxn,
                qcat, qcat, rcat,
                col(p0), col(q0), col(p1h), col(q1h), col(rr0), col(rr1))
    feats01, rsh0, rsh1 = unpack(acc01, npo1)

    (rel1, P1, Pn, Q1, Qn, Rf, p1s, q1s, pns, qns, rrf) = _tc_mid(
        BR, pcat, feats01, rsh0, rsh1, rel_p, W, a_out, a2_out)

    sc2, npo2 = _sc_edge_kernel(NP, RP, E1P // C, E2P // C, False)
    accf = sc2(idx1, idxn,
               Q1, Qn, Rf,
               col(p1s), col(q1s), col(pns), col(qns), col(rrf), col(rrf))
    featsf, rsf1, rsfn = unpack(accf, npo2)

    out_entity_final = _tc_post(BR, P1, Pn, featsf, rsf1, rsfn)
    return (out_entity_final[:N], rel1[:R])
